# Initial kernel scaffold; baseline (speedup 1.0000x reference)
import jax
import jax.numpy as jnp
from jax.experimental import pallas as pl


def kernel(x, e, u, edge_index, node2graph, norm_atom, norm_bond, params):
    raise NotImplementedError("write your pallas kernel here")


# speedup vs baseline: 1.3330x; 1.3330x over previous
import jax
import jax.numpy as jnp
from jax.experimental import pallas as pl

D = 128
NEG = -1e30


def _dotT(a, b):
    return jax.lax.dot_general(a, b, (((0,), (0,)), ((), ())),
                               preferred_element_type=jnp.float32)


def _mm(a, b):
    return jnp.dot(a, b, preferred_element_type=jnp.float32)


def _onehot(idx, g):
    return (idx == jax.lax.broadcasted_iota(jnp.int32, (idx.shape[0], g), 1)
            ).astype(jnp.float32)


def _emb_nodes_k(x_ref, u_ref, wx_ref, wu_ref, h_ref, ug_ref):
    h_ref[...] = _mm(x_ref[...], wx_ref[...])
    ug_ref[...] = _mm(u_ref[...], wu_ref[...])


def _emb_edges_k(e_ref, we_ref, eb_ref):
    eb_ref[...] = _mm(e_ref[...], we_ref[...])


def _edge_pre_k(hs_ref, hd_ref, eb_ref, e2g_ref, nb_ref, ug_ref,
                A_ref, B_ref, C_ref, Dg_ref, be_ref,
                t_ref, s1_ref, s2_ref):
    i = pl.program_id(0)
    oh = _onehot(e2g_ref[...], ug_ref.shape[0])
    uge = _mm(oh, ug_ref[...])
    t = (_mm(hs_ref[...], A_ref[...]) + _mm(hd_ref[...], B_ref[...])
         + _mm(eb_ref[...], C_ref[...]) + _mm(uge, Dg_ref[...])
         + be_ref[...]) * nb_ref[...]
    t_ref[...] = t

    @pl.when(i == 0)
    def _():
        s1_ref[...] = jnp.zeros_like(s1_ref)
        s2_ref[...] = jnp.zeros_like(s2_ref)

    s1_ref[...] += jnp.sum(t, axis=0, keepdims=True)
    s2_ref[...] += jnp.sum(t * t, axis=0, keepdims=True)


def _edge_post_k(t_ref, eb_ref, hs_ref, e2g_ref, s1_ref, s2_ref,
                 g_ref, b_ref, V_ref, ne_ref,
                 ebn_ref, sig_ref, msg_ref, seg_ref):
    i = pl.program_id(0)
    inv_e = 1.0 / ne_ref[0, 0]
    m = s1_ref[...] * inv_e
    var = s2_ref[...] * inv_e - m * m
    t = t_ref[...]
    en = g_ref[...] * (t - m) * jax.lax.rsqrt(var + 1e-5) + b_ref[...]
    en = jnp.maximum(en, 0.0)
    ebn = eb_ref[...] + en
    ebn_ref[...] = ebn
    sig = jax.nn.sigmoid(ebn)
    sig_ref[...] = sig
    msg_ref[...] = sig * _mm(hs_ref[...], V_ref[...])

    @pl.when(i == 0)
    def _():
        seg_ref[...] = jnp.zeros_like(seg_ref)

    oh = _onehot(e2g_ref[...], seg_ref.shape[0])
    seg_ref[...] += _dotT(oh, ebn)


def _node_k(h_ref, num_ref, den_ref, na_ref, n2g_ref, seg_e_ref,
            cn_ref, ce_ref, ug_ref,
            U_ref, bh_ref, g_ref, b_ref, Wg_ref, bg_ref,
            hn_ref, ugn_ref):
    h = h_ref[...]
    t = (_mm(h, U_ref[...]) + num_ref[...] / (den_ref[...] + 1e-6)
         + bh_ref[...]) * na_ref[...]
    m = jnp.mean(t, axis=0, keepdims=True)
    var = jnp.mean(t * t, axis=0, keepdims=True) - m * m
    hn = h + jnp.maximum(
        g_ref[...] * (t - m) * jax.lax.rsqrt(var + 1e-5) + b_ref[...], 0.0)
    hn_ref[...] = hn

    gsz = ug_ref.shape[0]
    oh = _onehot(n2g_ref[...], gsz)
    mean_h = _dotT(oh, hn) / cn_ref[...]
    mean_e = seg_e_ref[...] / ce_ref[...]
    ug = ug_ref[...]
    feats = jnp.concatenate([mean_h, mean_e, ug], axis=1)
    ugn_ref[...] = ug + jnp.maximum(_mm(feats, Wg_ref[...]) + bg_ref[...], 0.0)


def _s2s_k(h_ref, n2g_ref, ug_ref,
           wi0_ref, wh0_ref, b0_ref, wi1_ref, wh1_ref, b1_ref,
           wi2_ref, wh2_ref, b2_ref,
           w1_ref, fb1_ref, w2_ref, fb2_ref, w3_ref, fb3_ref,
           out_ref):
    h = h_ref[...]
    gsz = ug_ref.shape[0]
    oh = _onehot(n2g_ref[...], gsz)

    wi = [wi0_ref[...], wi1_ref[...], wi2_ref[...]]
    wh = [wh0_ref[...], wh1_ref[...], wh2_ref[...]]
    bs = [b0_ref[...], b1_ref[...], b2_ref[...]]

    qstar = jnp.zeros((gsz, 2 * D), jnp.float32)
    hs = [jnp.zeros((gsz, D), jnp.float32) for _ in range(3)]
    cs = [jnp.zeros((gsz, D), jnp.float32) for _ in range(3)]

    for _ in range(6):
        inp = qstar
        nh, nc = [], []
        for l in range(3):
            gates = _mm(inp, wi[l]) + _mm(hs[l], wh[l]) + bs[l]
            i_g = gates[:, :D]
            f_g = gates[:, D:2 * D]
            g_g = gates[:, 2 * D:3 * D]
            o_g = gates[:, 3 * D:]
            c = jax.nn.sigmoid(f_g) * cs[l] + jax.nn.sigmoid(i_g) * jnp.tanh(g_g)
            hh = jax.nn.sigmoid(o_g) * jnp.tanh(c)
            nh.append(hh)
            nc.append(c)
            inp = hh
        hs, cs = nh, nc
        q = hs[-1]

        qn = _mm(oh, q)
        logits = jnp.sum(h * qn, axis=-1, keepdims=True)
        masked = jnp.where(oh > 0.0, logits, NEG)
        mg = jnp.max(masked, axis=0, keepdims=True)
        mg = jnp.where(mg > 0.5 * NEG, mg, 0.0)
        m_node = jnp.sum(oh * mg, axis=1, keepdims=True)
        ex = jnp.exp(logits - m_node)
        den_g = jnp.sum(oh * ex, axis=0, keepdims=True) + 1e-12
        den_node = jnp.sum(oh * den_g, axis=1, keepdims=True)
        alpha = ex / den_node
        r = _dotT(oh, alpha * h)
        qstar = jnp.concatenate([q, r], axis=1)

    feats = jnp.concatenate([qstar, ug_ref[...]], axis=1)
    o = jnp.maximum(_mm(feats, w1_ref[...]) + fb1_ref[...], 0.0)
    o = jnp.maximum(_mm(o, w2_ref[...]) + fb2_ref[...], 0.0)
    out_ref[...] = _mm(o, w3_ref[...]) + fb3_ref[...]


def _pick_tile(e):
    for t in (3200, 2000, 1600, 1000, 800, 640, 500, 400, 320, 250, 200,
              160, 128, 100, 80, 64, 50, 40, 32, 25, 20, 16, 10, 8, 5, 4, 2, 1):
        if e % t == 0:
            return t
    return 1


def kernel(x, e, u, edge_index, node2graph, norm_atom, norm_bond, params):
    src = edge_index[0].astype(jnp.int32)
    dst = edge_index[1].astype(jnp.int32)
    N = x.shape[0]
    E = e.shape[0]
    G = u.shape[0]
    TE = _pick_tile(E)
    ntiles = E // TE

    n2g = node2graph.astype(jnp.int32).reshape(N, 1)
    e2g = jnp.take(node2graph.astype(jnp.int32), dst).reshape(E, 1)
    cnt_n = jnp.maximum(
        jax.ops.segment_sum(jnp.ones((N,), jnp.float32), node2graph, G),
        1.0).reshape(G, 1)
    cnt_e = jnp.maximum(
        jax.ops.segment_sum(jnp.ones((E,), jnp.float32), e2g[:, 0], G),
        1.0).reshape(G, 1)

    f32 = jnp.float32
    rep2 = lambda shape: pl.BlockSpec(shape, lambda i: (0, 0))
    tile = lambda w: pl.BlockSpec((TE, w), lambda i: (i, 0))

    h, ug = pl.pallas_call(
        _emb_nodes_k,
        out_shape=(jax.ShapeDtypeStruct((N, D), f32),
                   jax.ShapeDtypeStruct((G, D), f32)),
    )(x, u, params['emb']['Wx'], params['emb']['Wu'])

    eb = pl.pallas_call(
        _emb_edges_k,
        grid=(ntiles,),
        in_specs=[tile(16), rep2((16, D))],
        out_specs=tile(D),
        out_shape=jax.ShapeDtypeStruct((E, D), f32),
    )(e, params['emb']['We'])

    for lp in params['layers']:
        hs = jnp.take(h, src, axis=0)
        hd = jnp.take(h, dst, axis=0)

        t, s1, s2 = pl.pallas_call(
            _edge_pre_k,
            grid=(ntiles,),
            in_specs=[tile(D), tile(D), tile(D), tile(1), tile(1),
                      rep2((G, D)), rep2((D, D)), rep2((D, D)),
                      rep2((D, D)), rep2((D, D)), rep2((1, D))],
            out_specs=(tile(D), rep2((1, D)), rep2((1, D))),
            out_shape=(jax.ShapeDtypeStruct((E, D), f32),
                       jax.ShapeDtypeStruct((1, D), f32),
                       jax.ShapeDtypeStruct((1, D), f32)),
        )(hs, hd, eb, e2g, norm_bond, ug, lp['A'], lp['B'], lp['C'],
          lp['Dg'], lp['be'].reshape(1, D))

        ne = jnp.full((1, 1), float(E), f32)
        eb, sig, msg, seg_e = pl.pallas_call(
            _edge_post_k,
            grid=(ntiles,),
            in_specs=[tile(D), tile(D), tile(D), tile(1),
                      rep2((1, D)), rep2((1, D)), rep2((1, D)), rep2((1, D)),
                      rep2((D, D)), rep2((1, 1))],
            out_specs=(tile(D), tile(D), tile(D), rep2((G, D))),
            out_shape=(jax.ShapeDtypeStruct((E, D), f32),
                       jax.ShapeDtypeStruct((E, D), f32),
                       jax.ShapeDtypeStruct((E, D), f32),
                       jax.ShapeDtypeStruct((G, D), f32)),
        )(t, eb, hs, e2g, s1, s2, lp['bn_e_g'].reshape(1, D),
          lp['bn_e_b'].reshape(1, D), lp['V'], ne)

        num = jax.ops.segment_sum(msg, dst, N)
        den = jax.ops.segment_sum(sig, dst, N)

        h, ug = pl.pallas_call(
            _node_k,
            out_shape=(jax.ShapeDtypeStruct((N, D), f32),
                       jax.ShapeDtypeStruct((G, D), f32)),
        )(h, num, den, norm_atom, n2g, seg_e, cnt_n, cnt_e, ug,
          lp['U'], lp['bh'].reshape(1, D), lp['bn_h_g'].reshape(1, D),
          lp['bn_h_b'].reshape(1, D), lp['Wg'], lp['bg'].reshape(1, D))

    lw = params['lstm']
    fc = params['fc']
    out = pl.pallas_call(
        _s2s_k,
        out_shape=jax.ShapeDtypeStruct((G, 1), f32),
    )(h, n2g, ug,
      lw[0]['Wi'], lw[0]['Wh'], lw[0]['b'].reshape(1, 4 * D),
      lw[1]['Wi'], lw[1]['Wh'], lw[1]['b'].reshape(1, 4 * D),
      lw[2]['Wi'], lw[2]['Wh'], lw[2]['b'].reshape(1, 4 * D),
      fc['W1'], fc['b1'].reshape(1, D), fc['W2'], fc['b2'].reshape(1, 64),
      fc['W3'], fc['b3'].reshape(1, 1))
    return out
